# Initial kernel scaffold; baseline (speedup 1.0000x reference)
import jax
import jax.numpy as jnp
from jax.experimental import pallas as pl


def kernel(x_context, x_target, in_proj_w, in_proj_b, out_proj_w, out_proj_b, gate_w, gate_b, w1, b1, w2, b2, ln_c1_w, ln_c1_b, ln_c2_w, ln_c2_b, ln_t1_w, ln_t1_b, ln_t2_w, ln_t2_b):
    raise NotImplementedError("write your pallas kernel here")


# speedup vs baseline: 1.0317x; 1.0317x over previous
import functools

import jax
import jax.numpy as jnp
from jax.experimental import pallas as pl
from jax.experimental.pallas import tpu as pltpu

S = 2048
B = 1
D = 768
H = 12
DH = D // H
E = 8
TOPK = 2
DFF = 4 * D
N = S * B
P = N * TOPK
BT = 128
NW = P // BT + E
BM = 512


def _ln_body(x_ref, w_ref, b_ref, o_ref):
    x = x_ref[...]
    m = jnp.mean(x, axis=1, keepdims=True)
    c = x - m
    v = jnp.mean(c * c, axis=1, keepdims=True)
    o_ref[...] = c * jax.lax.rsqrt(v + 1e-5) * w_ref[...] + b_ref[...]


def _ln(x, w, b):
    return pl.pallas_call(
        _ln_body,
        grid=(S // BM,),
        in_specs=[
            pl.BlockSpec((BM, D), lambda i: (i, 0)),
            pl.BlockSpec((1, D), lambda i: (0, 0)),
            pl.BlockSpec((1, D), lambda i: (0, 0)),
        ],
        out_specs=pl.BlockSpec((BM, D), lambda i: (i, 0)),
        out_shape=jax.ShapeDtypeStruct((S, D), jnp.float32),
    )(x, w.reshape(1, D), b.reshape(1, D))


def _mm_body(x_ref, w_ref, b_ref, o_ref):
    o_ref[...] = (
        jnp.dot(x_ref[...], w_ref[...], preferred_element_type=jnp.float32)
        + b_ref[...]
    )


def _mm_res_body(x_ref, w_ref, b_ref, r_ref, o_ref):
    o_ref[...] = (
        jnp.dot(x_ref[...], w_ref[...], preferred_element_type=jnp.float32)
        + b_ref[...]
        + r_ref[...]
    )


def _mm(x, w_t, b, res=None):
    k, n = w_t.shape
    in_specs = [
        pl.BlockSpec((BM, k), lambda i: (i, 0)),
        pl.BlockSpec((k, n), lambda i: (0, 0)),
        pl.BlockSpec((1, n), lambda i: (0, 0)),
    ]
    args = [x, w_t, b.reshape(1, n)]
    body = _mm_body
    if res is not None:
        in_specs.append(pl.BlockSpec((BM, n), lambda i: (i, 0)))
        args.append(res)
        body = _mm_res_body
    return pl.pallas_call(
        body,
        grid=(S // BM,),
        in_specs=in_specs,
        out_specs=pl.BlockSpec((BM, n), lambda i: (i, 0)),
        out_shape=jax.ShapeDtypeStruct((S, n), jnp.float32),
    )(*args)


def _attn_body(q_ref, k_ref, v_ref, o_ref):
    q = q_ref[0]
    s = jax.lax.dot_general(
        q, k_ref[0], (((1,), (1,)), ((), ())),
        preferred_element_type=jnp.float32,
    ) * (1.0 / (DH ** 0.5))
    m = jnp.max(s, axis=1, keepdims=True)
    p = jnp.exp(s - m)
    l = jnp.sum(p, axis=1, keepdims=True)
    o = jax.lax.dot_general(
        p, v_ref[0], (((1,), (0,)), ((), ())),
        preferred_element_type=jnp.float32,
    )
    o_ref[0] = o / l


def _attn(qh, kh, vh):
    spec = pl.BlockSpec((1, S, DH), lambda h: (h, 0, 0))
    return pl.pallas_call(
        _attn_body,
        grid=(H,),
        in_specs=[spec, spec, spec],
        out_specs=spec,
        out_shape=jax.ShapeDtypeStruct((H, S, DH), jnp.float32),
    )(qh, kh, vh)


def _mha(q_in, kv_in, in_proj_w, in_proj_b, out_proj_w, out_proj_b, res):
    if q_in is kv_in:
        qkv = _mm(q_in, in_proj_w.T, in_proj_b)
    else:
        qt = _mm(q_in, in_proj_w[:D].T, in_proj_b[:D])
        kv = _mm(kv_in, in_proj_w[D:].T, in_proj_b[D:])
        qkv = jnp.concatenate([qt, kv], axis=1)
    def split(i):
        return (
            qkv[:, i * D:(i + 1) * D]
            .reshape(S, H, DH)
            .transpose(1, 0, 2)
        )
    o = _attn(split(0), split(1), split(2))
    o = o.transpose(1, 0, 2).reshape(S, D)
    return _mm(o, out_proj_w.T, out_proj_b, res=res)


def _gate_body(x_ref, gw_ref, gb_ref, idx_ref, val_ref, cnt_ref):
    logits = (
        jnp.dot(x_ref[...], gw_ref[...], preferred_element_type=jnp.float32)
        + gb_ref[...]
    )
    mx = jnp.max(logits, axis=1, keepdims=True)
    p = jnp.exp(logits - mx)
    p = p / jnp.sum(p, axis=1, keepdims=True)
    cnt_ref[...] = jnp.sum(p, axis=0, keepdims=True)
    lane = jax.lax.broadcasted_iota(jnp.int32, (S, E), 1)
    v1 = jnp.max(p, axis=1, keepdims=True)
    i1 = jnp.min(jnp.where(p == v1, lane, E), axis=1, keepdims=True)
    pm = jnp.where(lane == i1, -jnp.inf, p)
    v2 = jnp.max(pm, axis=1, keepdims=True)
    i2 = jnp.min(jnp.where(pm == v2, lane, E), axis=1, keepdims=True)
    den = v1 + v2
    val_ref[...] = jnp.where(
        lane == 0, v1 / den, jnp.where(lane == 1, v2 / den, 0.0)
    )
    idx_ref[...] = jnp.where(lane == 0, i1, jnp.where(lane == 1, i2, 0))


def _gate(x, gw_t, gb):
    full = pl.BlockSpec((S, E), lambda: (0, 0))
    idx, val, cnt = pl.pallas_call(
        _gate_body,
        in_specs=[
            pl.BlockSpec((S, D), lambda: (0, 0)),
            pl.BlockSpec((D, E), lambda: (0, 0)),
            pl.BlockSpec((1, E), lambda: (0, 0)),
        ],
        out_specs=[full, full, pl.BlockSpec((1, E), lambda: (0, 0))],
        out_shape=[
            jax.ShapeDtypeStruct((S, E), jnp.int32),
            jax.ShapeDtypeStruct((S, E), jnp.float32),
            jax.ShapeDtypeStruct((1, E), jnp.float32),
        ],
    )(x, gw_t, gb.reshape(1, E))
    return idx[:, :TOPK], val[:, :TOPK], cnt[0]


def _moe_body(eidx_ref, ids_ref, gate_ref, x_ref, w1_ref, b1_ref,
              w2_ref, b2_ref, r_ref, o_ref):
    w = pl.program_id(0)
    ids = ids_ref[0]
    gates = gate_ref[0]
    tok = jax.lax.broadcasted_iota(jnp.int32, (BT, S), 1)
    oh = (ids == tok).astype(jnp.float32)
    xs = jax.lax.dot_general(
        oh, x_ref[...], (((1,), (0,)), ((), ())),
        preferred_element_type=jnp.float32,
    )
    h = jnp.maximum(
        jax.lax.dot_general(
            xs, w1_ref[0], (((1,), (1,)), ((), ())),
            preferred_element_type=jnp.float32,
        ) + b1_ref[0],
        0.0,
    )
    y = jax.lax.dot_general(
        h, w2_ref[0], (((1,), (1,)), ((), ())),
        preferred_element_type=jnp.float32,
    ) + b2_ref[0]
    contrib = jax.lax.dot_general(
        oh * gates, y, (((0,), (0,)), ((), ())),
        preferred_element_type=jnp.float32,
    )

    @pl.when(w == 0)
    def _():
        o_ref[...] = r_ref[...] + contrib

    @pl.when(w > 0)
    def _():
        o_ref[...] += contrib


def _moe_ffn(x, expert_of_item, ids_tbl, gate_tbl, w1, b1, w2, b2, res):
    grid_spec = pltpu.PrefetchScalarGridSpec(
        num_scalar_prefetch=1,
        grid=(NW,),
        in_specs=[
            pl.BlockSpec((1, BT, 1), lambda w, e: (w, 0, 0)),
            pl.BlockSpec((1, BT, 1), lambda w, e: (w, 0, 0)),
            pl.BlockSpec((S, D), lambda w, e: (0, 0)),
            pl.BlockSpec((1, DFF, D), lambda w, e: (e[w], 0, 0)),
            pl.BlockSpec((1, 1, DFF), lambda w, e: (e[w], 0, 0)),
            pl.BlockSpec((1, D, DFF), lambda w, e: (e[w], 0, 0)),
            pl.BlockSpec((1, 1, D), lambda w, e: (e[w], 0, 0)),
            pl.BlockSpec((S, D), lambda w, e: (0, 0)),
        ],
        out_specs=pl.BlockSpec((S, D), lambda w, e: (0, 0)),
    )
    return pl.pallas_call(
        _moe_body,
        grid_spec=grid_spec,
        out_shape=jax.ShapeDtypeStruct((S, D), jnp.float32),
        compiler_params=pltpu.CompilerParams(
            vmem_limit_bytes=100 * 1024 * 1024),
    )(expert_of_item, ids_tbl, gate_tbl, x,
      w1, b1.reshape(E, 1, DFF), w2, b2.reshape(E, 1, D), res)


def _routing_tables(top_idx, top_val):
    e_flat = top_idx.reshape(-1)
    g_flat = top_val.reshape(-1)
    perm = jnp.argsort(e_flat, stable=True)
    sorted_tok = (perm // TOPK).astype(jnp.int32)
    sorted_gate = g_flat[perm]
    counts = jnp.sum(e_flat[None, :] == jnp.arange(E)[:, None], axis=1)
    offsets = jnp.concatenate([jnp.zeros((1,), jnp.int32),
                               jnp.cumsum(counts).astype(jnp.int32)])
    nitems = (counts + BT - 1) // BT
    cum_items = jnp.cumsum(nitems)
    c0 = jnp.concatenate([jnp.zeros((1,), jnp.int32),
                          cum_items.astype(jnp.int32)])
    ws = jnp.arange(NW)
    e_w = jnp.clip(jnp.searchsorted(cum_items, ws, side='right'), 0, E - 1)
    e_w = e_w.astype(jnp.int32)
    start = offsets[e_w] + (ws - c0[e_w]) * BT
    end = offsets[e_w + 1]
    rows = start[:, None] + jnp.arange(BT)[None, :]
    valid = rows < end[:, None]
    safe = jnp.clip(rows, 0, P - 1)
    ids_tbl = sorted_tok[safe].reshape(NW, BT, 1)
    gate_tbl = jnp.where(valid, sorted_gate[safe], 0.0).reshape(NW, BT, 1)
    return e_w, ids_tbl, gate_tbl


def _moe(x, gw_t, gb, w1, b1, w2, b2, res):
    top_idx, top_val, cnt = _gate(x, gw_t, gb)
    e_w, ids_tbl, gate_tbl = _routing_tables(top_idx, top_val)
    out = _moe_ffn(x, e_w, ids_tbl, gate_tbl, w1, b1, w2, b2, res)
    aux = E * jnp.sum((cnt / jnp.sum(cnt)) * (cnt / N))
    return out, aux


def kernel(x_context, x_target, in_proj_w, in_proj_b, out_proj_w,
           out_proj_b, gate_w, gate_b, w1, b1, w2, b2, ln_c1_w, ln_c1_b,
           ln_c2_w, ln_c2_b, ln_t1_w, ln_t1_b, ln_t2_w, ln_t2_b):
    xc0 = x_context.reshape(S, D)
    xt0 = x_target.reshape(S, D)
    gw_t = gate_w.T

    xcn = _ln(xc0, ln_c1_w, ln_c1_b)
    xc1 = _mha(xcn, xcn, in_proj_w, in_proj_b, out_proj_w, out_proj_b, xc0)
    xcn2 = _ln(xc1, ln_c2_w, ln_c2_b)
    xc2, aux1 = _moe(xcn2, gw_t, gate_b, w1, b1, w2, b2, xc1)

    xtn = _ln(xt0, ln_t1_w, ln_t1_b)
    xt1 = _mha(xtn, xc2, in_proj_w, in_proj_b, out_proj_w, out_proj_b, xt0)
    xtn2 = _ln(xt1, ln_t2_w, ln_t2_b)
    xt2, aux2 = _moe(xtn2, gw_t, gate_b, w1, b1, w2, b2, xt1)

    aux = 0.01 * (aux1 + aux2)
    return xc2.reshape(S, B, D), xt2.reshape(S, B, D), aux
